# Initial kernel scaffold; baseline (speedup 1.0000x reference)
import jax
import jax.numpy as jnp
from jax.experimental import pallas as pl


def kernel(x, edge_index, edge_weight, W1, att_src1, att_dst1, b1, W2, att_src2, att_dst2, b2):
    raise NotImplementedError("write your pallas kernel here")


# speedup vs baseline: 2.7366x; 2.7366x over previous
import jax
import jax.numpy as jnp
from jax.experimental import pallas as pl

_N = 10000
_E = 320000
_NODE_BLK = 2000
_EDGE_BLK = 2000


def _dense_kernel(x_ref, w_ref, asrc_ref, adst_ref, h_ref, s_ref, d_ref):
    h = jnp.dot(x_ref[...], w_ref[...], preferred_element_type=jnp.float32)
    h_ref[...] = h
    heads, ch = asrc_ref.shape
    h3 = h.reshape(h.shape[0], heads, ch)
    s_ref[...] = jnp.sum(h3 * asrc_ref[...][None, :, :], axis=-1)
    d_ref[...] = jnp.sum(h3 * adst_ref[...][None, :, :], axis=-1)


def _dense_elu_kernel(x_ref, b_ref, w_ref, asrc_ref, adst_ref, h_ref, s_ref, d_ref):
    xin = x_ref[...] + b_ref[...][None, :]
    xin = jnp.where(xin > 0, xin, jnp.exp(jnp.minimum(xin, 0.0)) - 1.0)
    h = jnp.dot(xin, w_ref[...], preferred_element_type=jnp.float32)
    h_ref[...] = h
    heads, ch = asrc_ref.shape
    h3 = h.reshape(h.shape[0], heads, ch)
    s_ref[...] = jnp.sum(h3 * asrc_ref[...][None, :, :], axis=-1)
    d_ref[...] = jnp.sum(h3 * adst_ref[...][None, :, :], axis=-1)


def _edge_e_kernel(s_ref, d_ref, e_ref):
    v = s_ref[...] + d_ref[...]
    e_ref[...] = jnp.where(v >= 0, v, 0.2 * v)


def _edge_ex_kernel(e_ref, m_ref, ex_ref):
    ex_ref[...] = jnp.exp(e_ref[...] - m_ref[...])


def _edge_msg_kernel(ex_ref, den_ref, hsrc_ref, ew_ref, msg_ref):
    alpha = ex_ref[...] / (den_ref[...] + 1e-16)
    heads = ex_ref.shape[1]
    rows, hc = hsrc_ref.shape
    ch = hc // heads
    msg = alpha[:, :, None] * hsrc_ref[...].reshape(rows, heads, ch)
    msg_ref[...] = msg.reshape(rows, hc) * ew_ref[...]


def _dense_pass(x, W, att_s, att_d, bias=None):
    n = x.shape[0]
    heads, ch = att_s.shape
    hc = heads * ch
    grid = (n // _NODE_BLK,)
    out_types = (
        jax.ShapeDtypeStruct((n, hc), jnp.float32),
        jax.ShapeDtypeStruct((n, heads), jnp.float32),
        jax.ShapeDtypeStruct((n, heads), jnp.float32),
    )
    full = lambda shape: pl.BlockSpec(shape, lambda i: (0, 0))
    row = lambda dim: pl.BlockSpec((_NODE_BLK, dim), lambda i: (i, 0))
    if bias is None:
        return pl.pallas_call(
            _dense_kernel,
            grid=grid,
            in_specs=[row(x.shape[1]), full(W.shape), full(att_s.shape), full(att_d.shape)],
            out_specs=(row(hc), row(heads), row(heads)),
            out_shape=out_types,
        )(x, W, att_s, att_d)
    vec = pl.BlockSpec((x.shape[1],), lambda i: (0,))
    return pl.pallas_call(
        _dense_elu_kernel,
        grid=grid,
        in_specs=[row(x.shape[1]), vec, full(W.shape), full(att_s.shape), full(att_d.shape)],
        out_specs=(row(hc), row(heads), row(heads)),
        out_shape=out_types,
    )(x, bias, W, att_s, att_d)


def _edge_call(kern, dims_in, dim_out, *arrays):
    grid = (_E // _EDGE_BLK,)
    specs = [pl.BlockSpec((_EDGE_BLK, d), lambda i: (i, 0)) for d in dims_in]
    return pl.pallas_call(
        kern,
        grid=grid,
        in_specs=specs,
        out_specs=pl.BlockSpec((_EDGE_BLK, dim_out), lambda i: (i, 0)),
        out_shape=jax.ShapeDtypeStruct((_E, dim_out), jnp.float32),
    )(*arrays)


def _gat_layer(x, src, dst, W, att_s, att_d, edge_weight, in_bias):
    heads, ch = att_s.shape
    hc = heads * ch
    h, a_s, a_d = _dense_pass(x, W, att_s, att_d, bias=in_bias)
    e = _edge_call(_edge_e_kernel, (heads, heads), heads,
                   jnp.take(a_s, src, axis=0), jnp.take(a_d, dst, axis=0))
    m = jax.ops.segment_max(e, dst, num_segments=_N)
    m = jnp.where(jnp.isfinite(m), m, 0.0)
    ex = _edge_call(_edge_ex_kernel, (heads, heads), heads,
                    e, jnp.take(m, dst, axis=0))
    denom = jax.ops.segment_sum(ex, dst, num_segments=_N)
    if edge_weight is None:
        ew = jnp.ones((_E, 1), jnp.float32)
    else:
        ew = edge_weight[:, None]
    msg = _edge_call(_edge_msg_kernel, (heads, heads, hc, 1), hc,
                     ex, jnp.take(denom, dst, axis=0), jnp.take(h, src, axis=0), ew)
    return jax.ops.segment_sum(msg, dst, num_segments=_N)


def kernel(x, edge_index, edge_weight, W1, att_src1, att_dst1, b1, W2, att_src2, att_dst2, b2):
    src = edge_index[0]
    dst = edge_index[1]
    raw1 = _gat_layer(x, src, dst, W1, att_src1, att_dst1, None, None)
    raw2 = _gat_layer(raw1, src, dst, W2, att_src2, att_dst2, edge_weight, b1)
    return raw2 + b2[None, :]
